# baseline (speedup 1.0000x reference)
import functools

import jax
import jax.numpy as jnp
from jax import lax
from jax.experimental import pallas as pl
from jax.experimental.pallas import tpu as pltpu
from jax.experimental.pallas import tpu_sc as plsc

EMB_DIM = 32
NEG_K = 20
NUM_CORES = 2
NUM_SUBCORES = 16
NUM_WORKERS = NUM_CORES * NUM_SUBCORES
CHUNK = 128
GROUP = 16
GATHER = 128


def _sc_dots(cen_idx, ctx_idx, neg_idx, syn0, syn1):
    B = cen_idx.shape[0]
    per_w = B // NUM_WORKERS
    n_chunks = per_w // CHUNK
    out_per_chunk = CHUNK * (1 + NEG_K)
    mesh = plsc.VectorSubcoreMesh(core_axis_name="c", subcore_axis_name="s")

    @functools.partial(
        pl.kernel,
        out_type=jax.ShapeDtypeStruct((B * (1 + NEG_K),), jnp.float32),
        mesh=mesh,
        scratch_types=[
            pltpu.VMEM((CHUNK,), jnp.int32),
            pltpu.VMEM((CHUNK,), jnp.int32),
            pltpu.VMEM((CHUNK * NEG_K,), jnp.int32),
            pltpu.VMEM((CHUNK, EMB_DIM), jnp.float32),
            pltpu.VMEM((CHUNK, EMB_DIM), jnp.float32),
            pltpu.VMEM((CHUNK * NEG_K, EMB_DIM), jnp.float32),
            pltpu.VMEM((CHUNK * (1 + NEG_K),), jnp.float32),
            pltpu.SemaphoreType.DMA,
        ],
        compiler_params=pltpu.CompilerParams(
            needs_layout_passes=False, use_tc_tiling_on_sc=False),
    )
    def sc_kernel(cen_hbm, ctx_hbm, neg_hbm, syn0_hbm, syn1_hbm, out_hbm,
                  cen_i, ctx_i, neg_i, cen_r, ctx_r, neg_r, ob, sem):
        wid = lax.axis_index("s") * NUM_CORES + lax.axis_index("c")
        iota = lax.iota(jnp.int32, GROUP)
        cols = [jnp.full((GROUP,), d, jnp.int32) for d in range(EMB_DIM)]

        def chunk_body(c, carry):
            base = wid * per_w + c * CHUNK
            pltpu.sync_copy(cen_hbm.at[pl.ds(base, CHUNK)], cen_i)
            pltpu.sync_copy(ctx_hbm.at[pl.ds(base, CHUNK)], ctx_i)
            pltpu.sync_copy(neg_hbm.at[pl.ds(base * NEG_K, CHUNK * NEG_K)], neg_i)
            copies = [
                pltpu.async_copy(syn0_hbm.at[cen_i], cen_r, sem),
                pltpu.async_copy(syn1_hbm.at[ctx_i], ctx_r, sem),
            ]
            for j in range(CHUNK * NEG_K // GATHER):
                copies.append(pltpu.async_copy(
                    syn1_hbm.at[neg_i.at[pl.ds(j * GATHER, GATHER)]],
                    neg_r.at[pl.ds(j * GATHER, GATHER)], sem))
            for cp in copies:
                cp.wait()

            def group_body(g, gcarry):
                e = g * GROUP + iota
                cen_d = [plsc.load_gather(cen_r, [e, cols[d]])
                         for d in range(EMB_DIM)]
                acc = cen_d[0] * plsc.load_gather(ctx_r, [e, cols[0]])
                for d in range(1, EMB_DIM):
                    acc = acc + cen_d[d] * plsc.load_gather(ctx_r, [e, cols[d]])
                ob[pl.ds(g * GROUP, GROUP)] = acc
                e_neg = e * NEG_K

                def neg_body(kk, kcarry):
                    row = e_neg + kk
                    acc_k = cen_d[0] * plsc.load_gather(neg_r, [row, cols[0]])
                    for d in range(1, EMB_DIM):
                        acc_k = acc_k + cen_d[d] * plsc.load_gather(
                            neg_r, [row, cols[d]])
                    ob[pl.ds(CHUNK + kk * CHUNK + g * GROUP, GROUP)] = -acc_k
                    return kcarry

                lax.fori_loop(0, NEG_K, neg_body, 0)
                return gcarry

            lax.fori_loop(0, CHUNK // GROUP, group_body, 0)
            pltpu.sync_copy(
                ob,
                out_hbm.at[pl.ds((wid * n_chunks + c) * out_per_chunk,
                                 out_per_chunk)])
            return carry

        lax.fori_loop(0, n_chunks, chunk_body, 0)

    return sc_kernel(cen_idx, ctx_idx, neg_idx, syn0, syn1)


def _tc_loss(dots):
    n = dots.shape[0]
    x2 = dots.reshape(n // 128, 128)

    def body(x_ref, o_ref):
        x = x_ref[...]
        ls = jnp.minimum(x, 0.0) - jnp.log1p(jnp.exp(-jnp.abs(x)))
        o_ref[0, 0] = -jnp.sum(jnp.sum(ls, axis=1))

    out = pl.pallas_call(
        body,
        out_shape=jax.ShapeDtypeStruct((1, 1), jnp.float32),
        out_specs=pl.BlockSpec(memory_space=pltpu.SMEM),
    )(x2)
    return out[0, 0]


def kernel(center_word, context_word, neg_sampling_words, syn0, syn1):
    cen = center_word.astype(jnp.int32)
    ctx = context_word.astype(jnp.int32)
    neg = neg_sampling_words.astype(jnp.int32).reshape(-1)
    dots = _sc_dots(cen, ctx, neg, syn0, syn1)
    return _tc_loss(dots)
